# Initial kernel scaffold; baseline (speedup 1.0000x reference)
import jax
import jax.numpy as jnp
from jax.experimental import pallas as pl


def kernel(x, edge_index, W1, b1, W2, b2, Wfc, bfc):
    raise NotImplementedError("write your pallas kernel here")


# speedup vs baseline: 20.5745x; 20.5745x over previous
import functools

import jax
import jax.numpy as jnp
from jax import lax
from jax.experimental import pallas as pl
from jax.experimental.pallas import tpu as pltpu
from jax.experimental.pallas import tpu_sc as plsc

N_NODES = 100000
N_EDGES = 1600000
N_PAD = 102400
EB = 2000
NBLK = N_EDGES // EB
NC = 2
NS = 16
NW = NC * NS
BLK_PER_W = NBLK // NW
RPT = N_PAD // NS
CH = 16
NCH = 4
EB2 = 1000
NBLK2 = N_EDGES // EB2
BLKW2 = NBLK2 // NW
RB = 1024
NTB = N_PAD // RB

_sc_mesh = functools.partial(
    plsc.VectorSubcoreMesh, core_axis_name="c", subcore_axis_name="s")


def _worker_id():
    return lax.axis_index("s") * NC + lax.axis_index("c")


def _sc_deg_body(dst_hbm, ones_hbm, zeros_hbm, out_hbm, ones_v, idx_v, acc_sh):
    c = lax.axis_index("c")
    s = lax.axis_index("s")
    w = _worker_id()
    pltpu.sync_copy(ones_hbm, ones_v)
    pltpu.sync_copy(zeros_hbm, acc_sh.at[pl.ds(s * RPT, RPT)])
    plsc.subcore_barrier()

    def body(j, carry):
        gblk = w + NW * j
        pltpu.sync_copy(dst_hbm.at[pl.ds(gblk * EB, EB)], idx_v)
        pltpu.sync_copy(ones_v, acc_sh.at[idx_v], add=True)
        return carry

    lax.fori_loop(0, BLK_PER_W, body, 0)
    plsc.subcore_barrier()
    pltpu.sync_copy(acc_sh.at[pl.ds(s * RPT, RPT)],
                    out_hbm.at[pl.ds(c * N_PAD + s * RPT, RPT)])


def _deg_call(dst):
    ones = jnp.ones((EB,), jnp.float32)
    zeros = jnp.zeros((RPT,), jnp.float32)
    return pl.kernel(
        _sc_deg_body,
        out_type=jax.ShapeDtypeStruct((NC * N_PAD,), jnp.float32),
        mesh=_sc_mesh(),
        compiler_params=pltpu.CompilerParams(use_tc_tiling_on_sc=False),
        scratch_types=[
            pltpu.VMEM((EB,), jnp.float32),
            pltpu.VMEM((EB,), jnp.int32),
            pltpu.VMEM_SHARED((N_PAD,), jnp.float32),
        ],
    )(dst, ones, zeros)


def _sc_seg4_body(src_hbm, dst_hbm, g1_hbm, zeros_hbm, out_hbm,
                  sidx_v, didx_v, rows_v, acc_sh, sem):
    c = lax.axis_index("c")
    s = lax.axis_index("s")
    w = _worker_id()
    pltpu.sync_copy(zeros_hbm, acc_sh.at[pl.ds(s * RPT, RPT)])
    plsc.subcore_barrier()

    def body(j, carry):
        gblk = w + NW * j
        pltpu.sync_copy(src_hbm.at[pl.ds(gblk * EB, EB)], sidx_v)
        pltpu.sync_copy(dst_hbm.at[pl.ds(gblk * EB, EB)], didx_v)
        pltpu.async_copy(g1_hbm.at[sidx_v], rows_v, sem).wait()
        pltpu.sync_copy(rows_v, acc_sh.at[didx_v], add=True)
        return carry

    lax.fori_loop(0, BLK_PER_W, body, 0)
    plsc.subcore_barrier()
    pltpu.sync_copy(acc_sh.at[pl.ds(s * RPT, RPT)],
                    out_hbm.at[pl.ds(c * N_PAD + s * RPT, RPT)])


def _seg4_call(src, dst, g1):
    zeros = jnp.zeros((RPT, 8), jnp.float32)
    return pl.kernel(
        _sc_seg4_body,
        out_type=jax.ShapeDtypeStruct((NC * N_PAD, 8), jnp.float32),
        mesh=_sc_mesh(),
        compiler_params=pltpu.CompilerParams(use_tc_tiling_on_sc=False),
        scratch_types=[
            pltpu.VMEM((EB,), jnp.int32),
            pltpu.VMEM((EB,), jnp.int32),
            pltpu.VMEM((EB, 8), jnp.float32),
            pltpu.VMEM_SHARED((N_PAD, 8), jnp.float32),
            pltpu.SemaphoreType.DMA,
        ],
    )(src, dst, g1, zeros)


def _sc_seg16_body(src_hbm, dst_hbm, g0_hbm, g1_hbm, g2_hbm, g3_hbm,
                   zeros_hbm, out_hbm, sidx_v, didx_v, rows_v, acc_sh, sem):
    c = lax.axis_index("c")
    s = lax.axis_index("s")
    w = _worker_id()
    tables = [g0_hbm, g1_hbm, g2_hbm, g3_hbm]
    for chunk in range(NCH):
        pltpu.sync_copy(zeros_hbm, acc_sh.at[pl.ds(s * RPT, RPT)])
        plsc.subcore_barrier()

        def body(j, carry, table=tables[chunk]):
            gblk = w + NW * j
            pltpu.sync_copy(src_hbm.at[pl.ds(gblk * EB2, EB2)], sidx_v)
            pltpu.sync_copy(dst_hbm.at[pl.ds(gblk * EB2, EB2)], didx_v)
            pltpu.async_copy(table.at[sidx_v], rows_v, sem).wait()
            pltpu.sync_copy(rows_v, acc_sh.at[didx_v], add=True)
            return carry

        lax.fori_loop(0, BLKW2, body, 0)
        plsc.subcore_barrier()
        base = (chunk * NC + c) * N_PAD + s * RPT
        pltpu.sync_copy(acc_sh.at[pl.ds(s * RPT, RPT)],
                        out_hbm.at[pl.ds(base, RPT)])
        plsc.subcore_barrier()


def _seg16_call(src, dst, g2s):
    zeros = jnp.zeros((RPT, CH), jnp.float32)
    return pl.kernel(
        _sc_seg16_body,
        out_type=jax.ShapeDtypeStruct((NCH * NC * N_PAD, CH), jnp.float32),
        mesh=_sc_mesh(),
        compiler_params=pltpu.CompilerParams(use_tc_tiling_on_sc=False),
        scratch_types=[
            pltpu.VMEM((EB2,), jnp.int32),
            pltpu.VMEM((EB2,), jnp.int32),
            pltpu.VMEM((EB2, CH), jnp.float32),
            pltpu.VMEM_SHARED((N_PAD, CH), jnp.float32),
            pltpu.SemaphoreType.DMA,
        ],
    )(src, dst, *g2s, zeros)


def _tc_prep_body(degp_ref, x_ref, dis_ref, g1_ref):
    deg = degp_ref[0] + degp_ref[1] + 1.0
    dis = lax.rsqrt(jnp.maximum(deg, 1.0))
    dis_ref[...] = dis
    g1 = dis * x_ref[...]
    g1_ref[...] = jnp.concatenate([g1, jnp.zeros_like(g1)], axis=1)


def _prep_call(degp, x_pad):
    return pl.pallas_call(
        _tc_prep_body,
        grid=(NTB,),
        in_specs=[
            pl.BlockSpec((NC, RB, 1), lambda i: (0, i, 0)),
            pl.BlockSpec((RB, 4), lambda i: (i, 0)),
        ],
        out_specs=[
            pl.BlockSpec((RB, 1), lambda i: (i, 0)),
            pl.BlockSpec((RB, 8), lambda i: (i, 0)),
        ],
        out_shape=[
            jax.ShapeDtypeStruct((N_PAD, 1), jnp.float32),
            jax.ShapeDtypeStruct((N_PAD, 8), jnp.float32),
        ],
    )(degp, x_pad)


def _tc_l1_body(seg_ref, g1_ref, dis_ref, w1_ref, b1_ref, *out_refs):
    dis = dis_ref[...]
    agg = dis * (seg_ref[0][:, :4] + seg_ref[1][:, :4] + g1_ref[:, :4])
    h1 = jnp.dot(agg, w1_ref[...], preferred_element_type=jnp.float32)
    h1 = jnp.maximum(h1 + b1_ref[...], 0.0)
    g2 = dis * h1
    for chunk in range(NCH):
        out_refs[chunk][...] = g2[:, chunk * CH:(chunk + 1) * CH]


def _l1_call(seg1, g1, dis, W1, b1):
    return pl.pallas_call(
        _tc_l1_body,
        grid=(NTB,),
        in_specs=[
            pl.BlockSpec((NC, RB, 8), lambda i: (0, i, 0)),
            pl.BlockSpec((RB, 8), lambda i: (i, 0)),
            pl.BlockSpec((RB, 1), lambda i: (i, 0)),
            pl.BlockSpec((4, 64), lambda i: (0, 0)),
            pl.BlockSpec((1, 64), lambda i: (0, 0)),
        ],
        out_specs=[pl.BlockSpec((RB, CH), lambda i: (i, 0))
                   for _ in range(NCH)],
        out_shape=[jax.ShapeDtypeStruct((N_PAD, CH), jnp.float32)
                   for _ in range(NCH)],
    )(seg1, g1, dis, W1, b1)


def _tc_l2_body(seg_ref, g0_ref, g1_ref, g2_ref, g3_ref, dis_ref,
                w2_ref, b2_ref, wfc_ref, bfc_ref, out_ref):
    dis = dis_ref[...]
    g_refs = [g0_ref, g1_ref, g2_ref, g3_ref]
    aggs = []
    for chunk in range(NCH):
        seg = seg_ref[chunk * NC] + seg_ref[chunk * NC + 1]
        aggs.append(dis * (seg + g_refs[chunk][...]))
    agg = jnp.concatenate(aggs, axis=1)
    h2 = jnp.dot(agg, w2_ref[...], preferred_element_type=jnp.float32)
    h2 = jnp.maximum(h2 + b2_ref[...], 0.0)
    out = jnp.dot(h2, wfc_ref[...], preferred_element_type=jnp.float32)
    out_ref[...] = out + bfc_ref[...]


def _l2_call(seg2, g2s, dis, W2, b2, Wfc, bfc):
    return pl.pallas_call(
        _tc_l2_body,
        grid=(NTB,),
        in_specs=[
            pl.BlockSpec((NCH * NC, RB, CH), lambda i: (0, i, 0)),
            pl.BlockSpec((RB, CH), lambda i: (i, 0)),
            pl.BlockSpec((RB, CH), lambda i: (i, 0)),
            pl.BlockSpec((RB, CH), lambda i: (i, 0)),
            pl.BlockSpec((RB, CH), lambda i: (i, 0)),
            pl.BlockSpec((RB, 1), lambda i: (i, 0)),
            pl.BlockSpec((64, 64), lambda i: (0, 0)),
            pl.BlockSpec((1, 64), lambda i: (0, 0)),
            pl.BlockSpec((64, 2), lambda i: (0, 0)),
            pl.BlockSpec((1, 2), lambda i: (0, 0)),
        ],
        out_specs=pl.BlockSpec((RB, 2), lambda i: (i, 0)),
        out_shape=jax.ShapeDtypeStruct((N_PAD, 2), jnp.float32),
    )(seg2, *g2s, dis, W2, b2, Wfc, bfc)


def kernel(x, edge_index, W1, b1, W2, b2, Wfc, bfc):
    src = edge_index[0]
    dst = edge_index[1]
    x_pad = jnp.pad(x, ((0, N_PAD - N_NODES), (0, 0)))

    degp = _deg_call(dst).reshape(NC, N_PAD, 1)
    dis, g1 = _prep_call(degp, x_pad)
    seg1 = _seg4_call(src, dst, g1).reshape(NC, N_PAD, 8)
    g2s = _l1_call(seg1, g1, dis, W1, b1.reshape(1, 64))
    seg2 = _seg16_call(src, dst, g2s).reshape(NCH * NC, N_PAD, CH)
    out = _l2_call(seg2, g2s, dis, W2, b2.reshape(1, 64),
                   Wfc, bfc.reshape(1, 2))
    return out[:N_NODES]


# speedup vs baseline: 22.6660x; 1.1017x over previous
import functools

import jax
import jax.numpy as jnp
from jax import lax
from jax.experimental import pallas as pl
from jax.experimental.pallas import tpu as pltpu
from jax.experimental.pallas import tpu_sc as plsc

N_NODES = 100000
N_EDGES = 1600000
N_PAD = 102400
NC = 2
NS = 16
NW = NC * NS
RPT = N_PAD // NS
CH = 16
NCH = 4
EBD = 5000
NBD = N_EDGES // EBD // NW
EB4 = 2000
NB4 = N_EDGES // EB4 // NW
EB2 = 400
NB2 = N_EDGES // EB2 // NW
RB = 1024
NTB = N_PAD // RB

_sc_mesh = functools.partial(
    plsc.VectorSubcoreMesh, core_axis_name="c", subcore_axis_name="s")


def _worker_id():
    return lax.axis_index("s") * NC + lax.axis_index("c")


def _edge_sweep(w, src_hbm, dst_hbm, table, acc_sh, sidx, didx, rows,
                semi, semg, sems, eb, nblocks):
    def fire_idx(b, p):
        off = (w + NW * b) * eb
        pltpu.async_copy(src_hbm.at[pl.ds(off, eb)], sidx[p], semi[p])
        pltpu.async_copy(dst_hbm.at[pl.ds(off, eb)], didx[p], semi[p])

    def wait_idx(p):
        pltpu.make_async_copy(src_hbm.at[pl.ds(0, eb)], sidx[p], semi[p]).wait()
        pltpu.make_async_copy(dst_hbm.at[pl.ds(0, eb)], didx[p], semi[p]).wait()

    def fire_scatter(p):
        if table is None:
            pltpu.async_copy(rows[0], acc_sh.at[didx[p]], sems[p], add=True)
        else:
            pltpu.async_copy(rows[p], acc_sh.at[didx[p]], sems[p], add=True)

    def wait_scatter(p):
        if table is None:
            pltpu.make_async_copy(rows[0], acc_sh.at[didx[p]], sems[p]).wait()
        else:
            pltpu.make_async_copy(rows[p], acc_sh.at[didx[p]], sems[p]).wait()

    def step(b, p, prefetch):
        wait_idx(p)
        if table is not None:
            gcopy = pltpu.async_copy(table.at[sidx[p]], rows[p], semg[p])

        if isinstance(b, int) and b > 0:
            wait_scatter(1 - p)
        elif not isinstance(b, int):
            @pl.when(b > 0)
            def _():
                wait_scatter(1 - p)

        if prefetch:
            @pl.when(b + 1 < nblocks)
            def _():
                fire_idx(b + 1, 1 - p)

        if table is not None:
            gcopy.wait()
        fire_scatter(p)

    fire_idx(0, 0)

    def dbl(i, carry):
        step(2 * i, 0, True)
        step(2 * i + 1, 1, True)
        return carry

    lax.fori_loop(0, nblocks // 2, dbl, 0)
    if nblocks % 2:
        step(nblocks - 1, 0, False)
    wait_scatter((nblocks - 1) % 2)


def _sc_deg_body(dst_hbm, ones_hbm, zeros_hbm, out_hbm,
                 ones_v, didx0, didx1, acc_sh, semi0, semi1, sems0, sems1):
    c = lax.axis_index("c")
    s = lax.axis_index("s")
    w = _worker_id()
    pltpu.sync_copy(ones_hbm, ones_v)
    pltpu.sync_copy(zeros_hbm, acc_sh.at[pl.ds(s * RPT, RPT)])
    plsc.subcore_barrier()
    _edge_sweep(w, dst_hbm, dst_hbm, None, acc_sh,
                [didx0, didx1], [didx0, didx1], [ones_v],
                [semi0, semi1], None, [sems0, sems1], EBD, NBD)
    plsc.subcore_barrier()
    pltpu.sync_copy(acc_sh.at[pl.ds(s * RPT, RPT)],
                    out_hbm.at[pl.ds(c * N_PAD + s * RPT, RPT)])


def _deg_call(dst):
    ones = jnp.ones((EBD,), jnp.float32)
    zeros = jnp.zeros((RPT,), jnp.float32)
    return pl.kernel(
        _sc_deg_body,
        out_type=jax.ShapeDtypeStruct((NC * N_PAD,), jnp.float32),
        mesh=_sc_mesh(),
        compiler_params=pltpu.CompilerParams(use_tc_tiling_on_sc=False),
        scratch_types=[
            pltpu.VMEM((EBD,), jnp.float32),
            pltpu.VMEM((EBD,), jnp.int32),
            pltpu.VMEM((EBD,), jnp.int32),
            pltpu.VMEM_SHARED((N_PAD,), jnp.float32),
            pltpu.SemaphoreType.DMA,
            pltpu.SemaphoreType.DMA,
            pltpu.SemaphoreType.DMA,
            pltpu.SemaphoreType.DMA,
        ],
    )(dst, ones, zeros)


def _sc_seg4_body(src_hbm, dst_hbm, g1_hbm, zeros_hbm, out_hbm,
                  sidx0, sidx1, didx0, didx1, rows0, rows1, acc_sh,
                  semi0, semi1, semg0, semg1, sems0, sems1):
    c = lax.axis_index("c")
    s = lax.axis_index("s")
    w = _worker_id()
    pltpu.sync_copy(zeros_hbm, acc_sh.at[pl.ds(s * RPT, RPT)])
    plsc.subcore_barrier()
    _edge_sweep(w, src_hbm, dst_hbm, g1_hbm, acc_sh,
                [sidx0, sidx1], [didx0, didx1], [rows0, rows1],
                [semi0, semi1], [semg0, semg1], [sems0, sems1], EB4, NB4)
    plsc.subcore_barrier()
    pltpu.sync_copy(acc_sh.at[pl.ds(s * RPT, RPT)],
                    out_hbm.at[pl.ds(c * N_PAD + s * RPT, RPT)])


def _seg4_call(src, dst, g1):
    zeros = jnp.zeros((RPT, 8), jnp.float32)
    return pl.kernel(
        _sc_seg4_body,
        out_type=jax.ShapeDtypeStruct((NC * N_PAD, 8), jnp.float32),
        mesh=_sc_mesh(),
        compiler_params=pltpu.CompilerParams(use_tc_tiling_on_sc=False),
        scratch_types=[
            pltpu.VMEM((EB4,), jnp.int32),
            pltpu.VMEM((EB4,), jnp.int32),
            pltpu.VMEM((EB4,), jnp.int32),
            pltpu.VMEM((EB4,), jnp.int32),
            pltpu.VMEM((EB4, 8), jnp.float32),
            pltpu.VMEM((EB4, 8), jnp.float32),
            pltpu.VMEM_SHARED((N_PAD, 8), jnp.float32),
            pltpu.SemaphoreType.DMA,
            pltpu.SemaphoreType.DMA,
            pltpu.SemaphoreType.DMA,
            pltpu.SemaphoreType.DMA,
            pltpu.SemaphoreType.DMA,
            pltpu.SemaphoreType.DMA,
        ],
    )(src, dst, g1, zeros)


def _sc_seg16_body(src_hbm, dst_hbm, g0_hbm, g1_hbm, g2_hbm, g3_hbm,
                   zeros_hbm, out_hbm,
                   sidx0, sidx1, didx0, didx1, rows0, rows1, acc_sh,
                   semi0, semi1, semg0, semg1, sems0, sems1):
    c = lax.axis_index("c")
    s = lax.axis_index("s")
    w = _worker_id()
    tables = [g0_hbm, g1_hbm, g2_hbm, g3_hbm]
    for chunk in range(NCH):
        pltpu.sync_copy(zeros_hbm, acc_sh.at[pl.ds(s * RPT, RPT)])
        plsc.subcore_barrier()
        _edge_sweep(w, src_hbm, dst_hbm, tables[chunk], acc_sh,
                    [sidx0, sidx1], [didx0, didx1], [rows0, rows1],
                    [semi0, semi1], [semg0, semg1], [sems0, sems1], EB2, NB2)
        plsc.subcore_barrier()
        base = (chunk * NC + c) * N_PAD + s * RPT
        pltpu.sync_copy(acc_sh.at[pl.ds(s * RPT, RPT)],
                        out_hbm.at[pl.ds(base, RPT)])
        plsc.subcore_barrier()


def _seg16_call(src, dst, g2s):
    zeros = jnp.zeros((RPT, CH), jnp.float32)
    return pl.kernel(
        _sc_seg16_body,
        out_type=jax.ShapeDtypeStruct((NCH * NC * N_PAD, CH), jnp.float32),
        mesh=_sc_mesh(),
        compiler_params=pltpu.CompilerParams(use_tc_tiling_on_sc=False),
        scratch_types=[
            pltpu.VMEM((EB2,), jnp.int32),
            pltpu.VMEM((EB2,), jnp.int32),
            pltpu.VMEM((EB2,), jnp.int32),
            pltpu.VMEM((EB2,), jnp.int32),
            pltpu.VMEM((EB2, CH), jnp.float32),
            pltpu.VMEM((EB2, CH), jnp.float32),
            pltpu.VMEM_SHARED((N_PAD, CH), jnp.float32),
            pltpu.SemaphoreType.DMA,
            pltpu.SemaphoreType.DMA,
            pltpu.SemaphoreType.DMA,
            pltpu.SemaphoreType.DMA,
            pltpu.SemaphoreType.DMA,
            pltpu.SemaphoreType.DMA,
        ],
    )(src, dst, *g2s, zeros)


def _tc_prep_body(degp_ref, x_ref, dis_ref, g1_ref):
    deg = degp_ref[0] + degp_ref[1] + 1.0
    dis = lax.rsqrt(jnp.maximum(deg, 1.0))
    dis_ref[...] = dis
    g1 = dis * x_ref[...]
    g1_ref[...] = jnp.concatenate([g1, jnp.zeros_like(g1)], axis=1)


def _prep_call(degp, x_pad):
    return pl.pallas_call(
        _tc_prep_body,
        grid=(NTB,),
        in_specs=[
            pl.BlockSpec((NC, RB, 1), lambda i: (0, i, 0)),
            pl.BlockSpec((RB, 4), lambda i: (i, 0)),
        ],
        out_specs=[
            pl.BlockSpec((RB, 1), lambda i: (i, 0)),
            pl.BlockSpec((RB, 8), lambda i: (i, 0)),
        ],
        out_shape=[
            jax.ShapeDtypeStruct((N_PAD, 1), jnp.float32),
            jax.ShapeDtypeStruct((N_PAD, 8), jnp.float32),
        ],
    )(degp, x_pad)


def _tc_l1_body(seg_ref, g1_ref, dis_ref, w1_ref, b1_ref, *out_refs):
    dis = dis_ref[...]
    agg = dis * (seg_ref[0][:, :4] + seg_ref[1][:, :4] + g1_ref[:, :4])
    h1 = jnp.dot(agg, w1_ref[...], preferred_element_type=jnp.float32)
    h1 = jnp.maximum(h1 + b1_ref[...], 0.0)
    g2 = dis * h1
    for chunk in range(NCH):
        out_refs[chunk][...] = g2[:, chunk * CH:(chunk + 1) * CH]


def _l1_call(seg1, g1, dis, W1, b1):
    return pl.pallas_call(
        _tc_l1_body,
        grid=(NTB,),
        in_specs=[
            pl.BlockSpec((NC, RB, 8), lambda i: (0, i, 0)),
            pl.BlockSpec((RB, 8), lambda i: (i, 0)),
            pl.BlockSpec((RB, 1), lambda i: (i, 0)),
            pl.BlockSpec((4, 64), lambda i: (0, 0)),
            pl.BlockSpec((1, 64), lambda i: (0, 0)),
        ],
        out_specs=[pl.BlockSpec((RB, CH), lambda i: (i, 0))
                   for _ in range(NCH)],
        out_shape=[jax.ShapeDtypeStruct((N_PAD, CH), jnp.float32)
                   for _ in range(NCH)],
    )(seg1, g1, dis, W1, b1)


def _tc_l2_body(seg_ref, g0_ref, g1_ref, g2_ref, g3_ref, dis_ref,
                w2_ref, b2_ref, wfc_ref, bfc_ref, out_ref):
    dis = dis_ref[...]
    g_refs = [g0_ref, g1_ref, g2_ref, g3_ref]
    aggs = []
    for chunk in range(NCH):
        seg = seg_ref[chunk * NC] + seg_ref[chunk * NC + 1]
        aggs.append(dis * (seg + g_refs[chunk][...]))
    agg = jnp.concatenate(aggs, axis=1)
    h2 = jnp.dot(agg, w2_ref[...], preferred_element_type=jnp.float32)
    h2 = jnp.maximum(h2 + b2_ref[...], 0.0)
    out = jnp.dot(h2, wfc_ref[...], preferred_element_type=jnp.float32)
    out_ref[...] = out + bfc_ref[...]


def _l2_call(seg2, g2s, dis, W2, b2, Wfc, bfc):
    return pl.pallas_call(
        _tc_l2_body,
        grid=(NTB,),
        in_specs=[
            pl.BlockSpec((NCH * NC, RB, CH), lambda i: (0, i, 0)),
            pl.BlockSpec((RB, CH), lambda i: (i, 0)),
            pl.BlockSpec((RB, CH), lambda i: (i, 0)),
            pl.BlockSpec((RB, CH), lambda i: (i, 0)),
            pl.BlockSpec((RB, CH), lambda i: (i, 0)),
            pl.BlockSpec((RB, 1), lambda i: (i, 0)),
            pl.BlockSpec((64, 64), lambda i: (0, 0)),
            pl.BlockSpec((1, 64), lambda i: (0, 0)),
            pl.BlockSpec((64, 2), lambda i: (0, 0)),
            pl.BlockSpec((1, 2), lambda i: (0, 0)),
        ],
        out_specs=pl.BlockSpec((RB, 2), lambda i: (i, 0)),
        out_shape=jax.ShapeDtypeStruct((N_PAD, 2), jnp.float32),
    )(seg2, *g2s, dis, W2, b2, Wfc, bfc)


def kernel(x, edge_index, W1, b1, W2, b2, Wfc, bfc):
    src = edge_index[0]
    dst = edge_index[1]
    x_pad = jnp.pad(x, ((0, N_PAD - N_NODES), (0, 0)))

    degp = _deg_call(dst).reshape(NC, N_PAD, 1)
    dis, g1 = _prep_call(degp, x_pad)
    seg1 = _seg4_call(src, dst, g1).reshape(NC, N_PAD, 8)
    g2s = _l1_call(seg1, g1, dis, W1, b1.reshape(1, 64))
    seg2 = _seg16_call(src, dst, g2s).reshape(NCH * NC, N_PAD, CH)
    out = _l2_call(seg2, g2s, dis, W2, b2.reshape(1, 64),
                   Wfc, bfc.reshape(1, 2))
    return out[:N_NODES]


# speedup vs baseline: 24.3780x; 1.0755x over previous
import functools

import jax
import jax.numpy as jnp
from jax import lax
from jax.experimental import pallas as pl
from jax.experimental.pallas import tpu as pltpu
from jax.experimental.pallas import tpu_sc as plsc

N_NODES = 100000
N_EDGES = 1600000
N_PAD = 102400
NC = 2
NS = 16
NW = NC * NS
RPT = N_PAD // NS
CH = 16
NCH = 4
EBD = 5000
NBD = N_EDGES // EBD // NW
EB4 = 2000
NB4 = N_EDGES // EB4 // NW
EB2 = 400
NB2 = N_EDGES // EB2 // NW
RB = 1024
NTB = N_PAD // RB

_sc_mesh = functools.partial(
    plsc.VectorSubcoreMesh, core_axis_name="c", subcore_axis_name="s")


def _worker_id():
    return lax.axis_index("s") * NC + lax.axis_index("c")


def _edge_sweep(w, src_hbm, dst_hbm, table, acc_sh, sidx, didx, rows,
                semi, semg, sems, eb, nblocks):
    def fire_idx(b, p):
        off = (w + NW * b) * eb
        pltpu.async_copy(src_hbm.at[pl.ds(off, eb)], sidx[p], semi[p])
        pltpu.async_copy(dst_hbm.at[pl.ds(off, eb)], didx[p], semi[p])

    def wait_idx(p):
        pltpu.make_async_copy(src_hbm.at[pl.ds(0, eb)], sidx[p], semi[p]).wait()
        pltpu.make_async_copy(dst_hbm.at[pl.ds(0, eb)], didx[p], semi[p]).wait()

    def fire_scatter(p):
        if table is None:
            pltpu.async_copy(rows[0], acc_sh.at[didx[p]], sems[p], add=True)
        else:
            pltpu.async_copy(rows[p], acc_sh.at[didx[p]], sems[p], add=True)

    def wait_scatter(p):
        if table is None:
            pltpu.make_async_copy(rows[0], acc_sh.at[didx[p]], sems[p]).wait()
        else:
            pltpu.make_async_copy(rows[p], acc_sh.at[didx[p]], sems[p]).wait()

    def step(b, p, prefetch):
        wait_idx(p)
        if table is not None:
            gcopy = pltpu.async_copy(table.at[sidx[p]], rows[p], semg[p])

        if isinstance(b, int) and b > 0:
            wait_scatter(1 - p)
        elif not isinstance(b, int):
            @pl.when(b > 0)
            def _():
                wait_scatter(1 - p)

        if prefetch:
            @pl.when(b + 1 < nblocks)
            def _():
                fire_idx(b + 1, 1 - p)

        if table is not None:
            gcopy.wait()
        fire_scatter(p)

    fire_idx(0, 0)

    def dbl(i, carry):
        step(2 * i, 0, True)
        step(2 * i + 1, 1, True)
        return carry

    lax.fori_loop(0, nblocks // 2, dbl, 0)
    if nblocks % 2:
        step(nblocks - 1, 0, False)
    wait_scatter((nblocks - 1) % 2)


def _sc_deg_body(dst_hbm, ones_hbm, zeros_hbm, out_hbm,
                 ones_v, didx0, didx1, acc_sh, semi0, semi1, sems0, sems1):
    c = lax.axis_index("c")
    s = lax.axis_index("s")
    w = _worker_id()
    pltpu.sync_copy(ones_hbm, ones_v)
    pltpu.sync_copy(zeros_hbm, acc_sh.at[pl.ds(s * RPT, RPT)])
    plsc.subcore_barrier()
    _edge_sweep(w, dst_hbm, dst_hbm, None, acc_sh,
                [didx0, didx1], [didx0, didx1], [ones_v],
                [semi0, semi1], None, [sems0, sems1], EBD, NBD)
    plsc.subcore_barrier()
    pltpu.sync_copy(acc_sh.at[pl.ds(s * RPT, RPT)],
                    out_hbm.at[pl.ds(c * N_PAD + s * RPT, RPT)])


def _deg_call(dst):
    ones = jnp.ones((EBD,), jnp.float32)
    zeros = jnp.zeros((RPT,), jnp.float32)
    return pl.kernel(
        _sc_deg_body,
        out_type=jax.ShapeDtypeStruct((NC * N_PAD,), jnp.float32),
        mesh=_sc_mesh(),
        compiler_params=pltpu.CompilerParams(use_tc_tiling_on_sc=False),
        scratch_types=[
            pltpu.VMEM((EBD,), jnp.float32),
            pltpu.VMEM((EBD,), jnp.int32),
            pltpu.VMEM((EBD,), jnp.int32),
            pltpu.VMEM_SHARED((N_PAD,), jnp.float32),
            pltpu.SemaphoreType.DMA,
            pltpu.SemaphoreType.DMA,
            pltpu.SemaphoreType.DMA,
            pltpu.SemaphoreType.DMA,
        ],
    )(dst, ones, zeros)


def _sc_seg4_body(src_hbm, dst_hbm, g1_hbm, zeros_hbm, out_hbm,
                  sidx0, sidx1, didx0, didx1, rows0, rows1, acc_sh,
                  semi0, semi1, semg0, semg1, sems0, sems1):
    c = lax.axis_index("c")
    s = lax.axis_index("s")
    w = _worker_id()
    pltpu.sync_copy(zeros_hbm, acc_sh.at[pl.ds(s * RPT, RPT)])
    plsc.subcore_barrier()
    _edge_sweep(w, src_hbm, dst_hbm, g1_hbm, acc_sh,
                [sidx0, sidx1], [didx0, didx1], [rows0, rows1],
                [semi0, semi1], [semg0, semg1], [sems0, sems1], EB4, NB4)
    plsc.subcore_barrier()
    pltpu.sync_copy(acc_sh.at[pl.ds(s * RPT, RPT)],
                    out_hbm.at[pl.ds(s * RPT, RPT), pl.ds(8 * c, 8)])


def _seg4_call(src, dst, g1):
    zeros = jnp.zeros((RPT, 8), jnp.float32)
    return pl.kernel(
        _sc_seg4_body,
        out_type=jax.ShapeDtypeStruct((N_PAD, 16), jnp.float32),
        mesh=_sc_mesh(),
        compiler_params=pltpu.CompilerParams(use_tc_tiling_on_sc=False),
        scratch_types=[
            pltpu.VMEM((EB4,), jnp.int32),
            pltpu.VMEM((EB4,), jnp.int32),
            pltpu.VMEM((EB4,), jnp.int32),
            pltpu.VMEM((EB4,), jnp.int32),
            pltpu.VMEM((EB4, 8), jnp.float32),
            pltpu.VMEM((EB4, 8), jnp.float32),
            pltpu.VMEM_SHARED((N_PAD, 8), jnp.float32),
            pltpu.SemaphoreType.DMA,
            pltpu.SemaphoreType.DMA,
            pltpu.SemaphoreType.DMA,
            pltpu.SemaphoreType.DMA,
            pltpu.SemaphoreType.DMA,
            pltpu.SemaphoreType.DMA,
        ],
    )(src, dst, g1, zeros)


def _sc_seg16_body(src_hbm, dst_hbm, g0_hbm, g1_hbm, g2_hbm, g3_hbm,
                   zeros_hbm, out_hbm,
                   sidx0, sidx1, didx0, didx1, rows0, rows1, acc_sh,
                   semi0, semi1, semg0, semg1, sems0, sems1):
    c = lax.axis_index("c")
    s = lax.axis_index("s")
    w = _worker_id()
    tables = [g0_hbm, g1_hbm, g2_hbm, g3_hbm]
    for chunk in range(NCH):
        pltpu.sync_copy(zeros_hbm, acc_sh.at[pl.ds(s * RPT, RPT)])
        plsc.subcore_barrier()
        _edge_sweep(w, src_hbm, dst_hbm, tables[chunk], acc_sh,
                    [sidx0, sidx1], [didx0, didx1], [rows0, rows1],
                    [semi0, semi1], [semg0, semg1], [sems0, sems1], EB2, NB2)
        plsc.subcore_barrier()
        base = c * N_PAD + s * RPT
        pltpu.sync_copy(acc_sh.at[pl.ds(s * RPT, RPT)],
                        out_hbm.at[pl.ds(base, RPT), pl.ds(CH * chunk, CH)])
        plsc.subcore_barrier()


def _seg16_call(src, dst, g2s):
    zeros = jnp.zeros((RPT, CH), jnp.float32)
    return pl.kernel(
        _sc_seg16_body,
        out_type=jax.ShapeDtypeStruct((NC * N_PAD, 64), jnp.float32),
        mesh=_sc_mesh(),
        compiler_params=pltpu.CompilerParams(use_tc_tiling_on_sc=False),
        scratch_types=[
            pltpu.VMEM((EB2,), jnp.int32),
            pltpu.VMEM((EB2,), jnp.int32),
            pltpu.VMEM((EB2,), jnp.int32),
            pltpu.VMEM((EB2,), jnp.int32),
            pltpu.VMEM((EB2, CH), jnp.float32),
            pltpu.VMEM((EB2, CH), jnp.float32),
            pltpu.VMEM_SHARED((N_PAD, CH), jnp.float32),
            pltpu.SemaphoreType.DMA,
            pltpu.SemaphoreType.DMA,
            pltpu.SemaphoreType.DMA,
            pltpu.SemaphoreType.DMA,
            pltpu.SemaphoreType.DMA,
            pltpu.SemaphoreType.DMA,
        ],
    )(src, dst, *g2s, zeros)


def _tc_prep_body(degp_ref, x_ref, dis_ref, g1_ref):
    deg = degp_ref[0] + degp_ref[1] + 1.0
    dis = lax.rsqrt(jnp.maximum(deg, 1.0))
    dis_ref[...] = dis
    g1 = dis * x_ref[...]
    g1_ref[...] = jnp.concatenate([g1, jnp.zeros_like(g1)], axis=1)


def _prep_call(degp, x_pad):
    return pl.pallas_call(
        _tc_prep_body,
        grid=(NTB,),
        in_specs=[
            pl.BlockSpec((NC, RB, 1), lambda i: (0, i, 0)),
            pl.BlockSpec((RB, 4), lambda i: (i, 0)),
        ],
        out_specs=[
            pl.BlockSpec((RB, 1), lambda i: (i, 0)),
            pl.BlockSpec((RB, 8), lambda i: (i, 0)),
        ],
        out_shape=[
            jax.ShapeDtypeStruct((N_PAD, 1), jnp.float32),
            jax.ShapeDtypeStruct((N_PAD, 8), jnp.float32),
        ],
    )(degp, x_pad)


def _tc_l1_body(seg_ref, g1_ref, dis_ref, w1_ref, b1_ref, *out_refs):
    dis = dis_ref[...]
    agg = dis * (seg_ref[:, 0:4] + seg_ref[:, 8:12] + g1_ref[:, :4])
    h1 = jnp.dot(agg, w1_ref[...], preferred_element_type=jnp.float32)
    h1 = jnp.maximum(h1 + b1_ref[...], 0.0)
    g2 = dis * h1
    for chunk in range(NCH):
        out_refs[chunk][...] = g2[:, chunk * CH:(chunk + 1) * CH]
    out_refs[NCH][...] = dis * g2


def _l1_call(seg1, g1, dis, W1, b1):
    return pl.pallas_call(
        _tc_l1_body,
        grid=(NTB,),
        in_specs=[
            pl.BlockSpec((RB, 16), lambda i: (i, 0)),
            pl.BlockSpec((RB, 8), lambda i: (i, 0)),
            pl.BlockSpec((RB, 1), lambda i: (i, 0)),
            pl.BlockSpec((4, 64), lambda i: (0, 0)),
            pl.BlockSpec((1, 64), lambda i: (0, 0)),
        ],
        out_specs=[pl.BlockSpec((RB, CH), lambda i: (i, 0))
                   for _ in range(NCH)] + [pl.BlockSpec((RB, 64), lambda i: (i, 0))],
        out_shape=[jax.ShapeDtypeStruct((N_PAD, CH), jnp.float32)
                   for _ in range(NCH)] + [jax.ShapeDtypeStruct((N_PAD, 64), jnp.float32)],
    )(seg1, g1, dis, W1, b1)


def _tc_l2_body(seg_ref, z_ref, dis_ref,
                w2_ref, b2_ref, wfc_ref, bfc_ref, out_ref):
    dis = dis_ref[...]
    agg = dis * (seg_ref[0] + seg_ref[1]) + z_ref[...]
    h2 = jnp.dot(agg, w2_ref[...], preferred_element_type=jnp.float32)
    h2 = jnp.maximum(h2 + b2_ref[...], 0.0)
    out = jnp.dot(h2, wfc_ref[...], preferred_element_type=jnp.float32)
    out_ref[...] = out + bfc_ref[...]


def _l2_call(seg2, z, dis, W2, b2, Wfc, bfc):
    return pl.pallas_call(
        _tc_l2_body,
        grid=(NTB,),
        in_specs=[
            pl.BlockSpec((NC, RB, 64), lambda i: (0, i, 0)),
            pl.BlockSpec((RB, 64), lambda i: (i, 0)),
            pl.BlockSpec((RB, 1), lambda i: (i, 0)),
            pl.BlockSpec((64, 64), lambda i: (0, 0)),
            pl.BlockSpec((1, 64), lambda i: (0, 0)),
            pl.BlockSpec((64, 2), lambda i: (0, 0)),
            pl.BlockSpec((1, 2), lambda i: (0, 0)),
        ],
        out_specs=pl.BlockSpec((RB, 2), lambda i: (i, 0)),
        out_shape=jax.ShapeDtypeStruct((N_PAD, 2), jnp.float32),
    )(seg2, z, dis, W2, b2, Wfc, bfc)


def kernel(x, edge_index, W1, b1, W2, b2, Wfc, bfc):
    src = edge_index[0]
    dst = edge_index[1]
    x_pad = jnp.pad(x, ((0, N_PAD - N_NODES), (0, 0)))

    degp = _deg_call(dst).reshape(NC, N_PAD, 1)
    dis, g1 = _prep_call(degp, x_pad)
    seg1 = _seg4_call(src, dst, g1)
    *g2s, z = _l1_call(seg1, g1, dis, W1, b1.reshape(1, 64))
    seg2 = _seg16_call(src, dst, g2s).reshape(NC, N_PAD, 64)
    out = _l2_call(seg2, z, dis, W2, b2.reshape(1, 64),
                   Wfc, bfc.reshape(1, 2))
    return out[:N_NODES]
